# baseline (speedup 1.0000x reference)
import functools

import jax
import jax.numpy as jnp
from jax import lax
from jax.experimental import pallas as pl
from jax.experimental.pallas import tpu as pltpu
from jax.experimental.pallas import tpu_sc as plsc

_N = 10000
_E = 320000
_D = 128
_DE = 16
_H = 128
_G = 64
_RH = 500

_NC = 2
_NS = 16
_CHUNK = 40
_EPT = _E // (_NC * _NS)
_NCHUNK = _EPT // _CHUNK
_IR = 8
_FCH = 40
_NF = _N // _FCH
_NFT = -(-_NF // _NS)


def _sc_edge_body(x_hbm, e_hbm, idx_hbm, part_hbm,
                  ibuf, e_buf, xg_buf, msg_buf,
                  aggr_sh, isem, esem, gsem, ssem):
    c = lax.axis_index("c")
    s = lax.axis_index("s")

    def _zrow(i, carry):
        for j in range(_D // 16):
            msg_buf[0, i, pl.ds(j * 16, 16)] = jnp.zeros((16,), jnp.float32)
        return carry
    lax.fori_loop(0, _FCH, _zrow, 0)
    for z in range(_NFT):
        k = z * _NS + s
        @pl.when(k < _NF)
        def _():
            pltpu.sync_copy(msg_buf.at[0], aggr_sh.at[pl.ds(k * _FCH, _FCH)])
    plsc.subcore_barrier()

    t = c * _NS + s
    base = t * _EPT

    def _load_idx(j):
        pltpu.async_copy(idx_hbm.at[t, j], ibuf.at[j % _IR], isem.at[j % _IR])

    def _start_loads(j, b):
        pltpu.make_async_copy(idx_hbm.at[t, j], ibuf.at[j % _IR],
                              isem.at[j % _IR]).wait()
        pltpu.async_copy(e_hbm.at[pl.ds(base + j * _CHUNK, _CHUNK)],
                         e_buf.at[b], esem.at[b])
        pltpu.async_copy(x_hbm.at[ibuf.at[j % _IR, 0]], xg_buf.at[b],
                         gsem.at[b])

    for j in range(6):
        _load_idx(j)
    for b in range(2):
        _start_loads(b, b)

    def _outer(gi, carry):
        g = gi * 2
        for b in range(2):
            j = g + b
            pltpu.make_async_copy(
                e_hbm.at[pl.ds(base + j * _CHUNK, _CHUNK)],
                e_buf.at[b], esem.at[b]).wait()
            pltpu.make_async_copy(
                x_hbm.at[ibuf.at[j % _IR, 0]], xg_buf.at[b], gsem.at[b]).wait()
            @pl.when(j >= 2)
            def _():
                pltpu.make_async_copy(
                    msg_buf.at[b], aggr_sh.at[ibuf.at[(j + _IR - 2) % _IR, 1]],
                    ssem.at[b]).wait()

            def _row(i, carry2):
                for j2 in range(_D // 16):
                    sl = pl.ds(j2 * 16, 16)
                    v = xg_buf[b, i, sl] + e_buf[b, i, sl]
                    msg_buf[b, i, sl] = jnp.maximum(v, 0.0)
                return carry2
            lax.fori_loop(0, _CHUNK, _row, 0)
            pltpu.async_copy(msg_buf.at[b], aggr_sh.at[ibuf.at[j % _IR, 1]],
                             ssem.at[b], add=True)
            @pl.when(j + 2 < _NCHUNK)
            def _():
                _start_loads(j + 2, b)
            @pl.when(j + 6 < _NCHUNK)
            def _():
                _load_idx(j + 6)
        return carry
    lax.fori_loop(0, _NCHUNK // 2, _outer, 0)
    for j in (_NCHUNK - 2, _NCHUNK - 1):
        b = j % 2
        pltpu.make_async_copy(msg_buf.at[b], aggr_sh.at[ibuf.at[j % _IR, 1]],
                              ssem.at[b]).wait()
    plsc.subcore_barrier()

    for z in range(_NFT):
        k = z * _NS + s
        @pl.when(k < _NF)
        def _():
            r0 = k * _FCH
            pltpu.sync_copy(aggr_sh.at[pl.ds(r0, _FCH)], msg_buf.at[0])
            pltpu.sync_copy(msg_buf.at[0], part_hbm.at[pl.ds(c * _N + r0, _FCH)])


@functools.cache
def _sc_edge_kernel():
    return pl.kernel(
        _sc_edge_body,
        out_type=jax.ShapeDtypeStruct((2 * _N, _D), jnp.float32),
        mesh=plsc.VectorSubcoreMesh(core_axis_name="c", subcore_axis_name="s",
                                    num_cores=_NC, num_subcores=_NS),
        scratch_types=[
            pltpu.VMEM((_IR, 2, _CHUNK), jnp.int32),
            pltpu.VMEM((2, _CHUNK, _D), jnp.float32),
            pltpu.VMEM((2, _CHUNK, _D), jnp.float32),
            pltpu.VMEM((2, _CHUNK, _D), jnp.float32),
            pltpu.VMEM_SHARED((_N, _D), jnp.float32),
            pltpu.SemaphoreType.DMA((_IR,)),
            pltpu.SemaphoreType.DMA((2,)),
            pltpu.SemaphoreType.DMA((2,)),
            pltpu.SemaphoreType.DMA((2,)),
        ],
    )


def _sc_edge(x, e, idx):
    return _sc_edge_kernel()(x, e, idx)


_BE = 8000


def _e_body(ea_ref, w_ref, b_ref, o_ref):
    o_ref[...] = (jnp.dot(ea_ref[...], w_ref[...],
                          preferred_element_type=jnp.float32) + b_ref[...])


def _edge_embed(edge_attr, eW, eb):
    return pl.pallas_call(
        _e_body,
        grid=(_E // _BE,),
        in_specs=[
            pl.BlockSpec((_BE, _DE), lambda i: (i, 0)),
            pl.BlockSpec((_DE, _D), lambda i: (0, 0)),
            pl.BlockSpec((1, _D), lambda i: (0, 0)),
        ],
        out_specs=pl.BlockSpec((_BE, _D), lambda i: (i, 0)),
        out_shape=jax.ShapeDtypeStruct((_E, _D), jnp.float32),
    )(edge_attr, eW, eb)


def _layer_body(x_ref, part_ref, w1_ref, b1_ref, g_ref, bb_ref,
                w2_ref, b2_ref, o_ref):
    h = x_ref[...] + part_ref[0] + part_ref[1]
    t = jnp.dot(h, w1_ref[...], preferred_element_type=jnp.float32) + b1_ref[...]
    m = jnp.mean(t, axis=0, keepdims=True)
    v = jnp.mean(t * t, axis=0, keepdims=True) - m * m
    t = (t - m) * lax.rsqrt(v + 1e-5) * g_ref[...] + bb_ref[...]
    t = jnp.where(t > 0, t, 0.01 * t)
    t = jnp.dot(t, w2_ref[...], preferred_element_type=jnp.float32) + b2_ref[...]
    t = jnp.where(t > 0, t, 0.01 * t)
    t = jnp.where(t > 0, t, 0.01 * t)
    o_ref[...] = t


def _layer_tc(x, part2, p):
    return pl.pallas_call(
        _layer_body,
        out_shape=jax.ShapeDtypeStruct((_N, _H), jnp.float32),
    )(x, part2, p['W1'], p['b1'][None], p['bn_g'][None], p['bn_b'][None],
      p['W2'], p['b2'][None])


def _head_body(h_ref, batch_ref, wr_ref, br_ref, we_ref, be_ref, o_ref):
    gid = lax.broadcasted_iota(jnp.int32, (_G, 1), 0)
    onehot = (batch_ref[...] == gid).astype(jnp.float32)
    sums = jnp.dot(onehot, h_ref[...], preferred_element_type=jnp.float32)
    cnt = jnp.sum(onehot, axis=1, keepdims=True)
    pooled = sums / jnp.maximum(cnt, 1.0)
    t = jnp.dot(pooled, wr_ref[...], preferred_element_type=jnp.float32) + br_ref[...]
    t = jnp.where(t > 0, t, 0.01 * t)
    o_ref[...] = jnp.dot(t, we_ref[...], preferred_element_type=jnp.float32) + be_ref[...]


def _head_tc(h, batch, p):
    return pl.pallas_call(
        _head_body,
        out_shape=jax.ShapeDtypeStruct((_G, 1), jnp.float32),
    )(h, batch[None], p['Wr1'], p['br1'][None], p['We'], p['be'][None])


def kernel(x, edge_attr, edge_index, batch, params):
    idx = jnp.stack([edge_index[0].reshape(_NC * _NS, _NCHUNK, _CHUNK),
                     edge_index[1].reshape(_NC * _NS, _NCHUNK, _CHUNK)],
                    axis=2)
    layers = params['layers']
    h = x
    for i in range(3):
        e_i = _edge_embed(edge_attr, layers[i]['eW'], layers[i]['eb'][None])
        part = _sc_edge(h, e_i, idx)
        h = _layer_tc(h, part.reshape(2, _N, _D), layers[i])
    return _head_tc(h, batch, params)


# speedup vs baseline: 1.0462x; 1.0462x over previous
import functools

import jax
import jax.numpy as jnp
from jax import lax
from jax.experimental import pallas as pl
from jax.experimental.pallas import tpu as pltpu
from jax.experimental.pallas import tpu_sc as plsc

_N = 10000
_E = 320000
_D = 128
_DE = 16
_H = 128
_G = 64
_RH = 500

_NC = 2
_NS = 16
_CHUNK = 40
_EPT = _E // (_NC * _NS)
_NCHUNK = _EPT // _CHUNK
_IR = 8
_FCH = 40
_NF = _N // _FCH
_NFT = -(-_NF // _NS)


def _sc_edge_body(x_hbm, e_hbm, idx_hbm, part_hbm,
                  ibuf, e_buf, xg_buf, msg_buf,
                  aggr_sh, isem, esem, gsem, ssem, fsem):
    c = lax.axis_index("c")
    s = lax.axis_index("s")

    def _zrow(i, carry):
        for j in range(_D // 16):
            msg_buf[0, i, pl.ds(j * 16, 16)] = jnp.zeros((16,), jnp.float32)
        return carry
    lax.fori_loop(0, _FCH, _zrow, 0)
    for z in range(_NFT):
        k = z * _NS + s
        @pl.when(k < _NF)
        def _():
            pltpu.async_copy(msg_buf.at[0], aggr_sh.at[pl.ds(k * _FCH, _FCH)],
                             fsem.at[z])
    for z in range(_NFT):
        k = z * _NS + s
        @pl.when(k < _NF)
        def _():
            pltpu.make_async_copy(
                msg_buf.at[0], aggr_sh.at[pl.ds(k * _FCH, _FCH)],
                fsem.at[z]).wait()
    plsc.subcore_barrier()

    t = c * _NS + s
    base = t * _EPT

    def _load_idx(j):
        pltpu.async_copy(idx_hbm.at[t, j], ibuf.at[j % _IR], isem.at[j % _IR])

    def _start_loads(j, b):
        pltpu.make_async_copy(idx_hbm.at[t, j], ibuf.at[j % _IR],
                              isem.at[j % _IR]).wait()
        pltpu.async_copy(e_hbm.at[pl.ds(base + j * _CHUNK, _CHUNK)],
                         e_buf.at[b], esem.at[b])
        pltpu.async_copy(x_hbm.at[ibuf.at[j % _IR, 0]], xg_buf.at[b],
                         gsem.at[b])

    for j in range(6):
        _load_idx(j)
    for b in range(2):
        _start_loads(b, b)

    def _outer(gi, carry):
        g = gi * 2
        for b in range(2):
            j = g + b
            pltpu.make_async_copy(
                e_hbm.at[pl.ds(base + j * _CHUNK, _CHUNK)],
                e_buf.at[b], esem.at[b]).wait()
            pltpu.make_async_copy(
                x_hbm.at[ibuf.at[j % _IR, 0]], xg_buf.at[b], gsem.at[b]).wait()
            @pl.when(j >= 2)
            def _():
                pltpu.make_async_copy(
                    msg_buf.at[b], aggr_sh.at[ibuf.at[(j + _IR - 2) % _IR, 1]],
                    ssem.at[b]).wait()

            def _row(i, carry2):
                for j2 in range(_D // 16):
                    sl = pl.ds(j2 * 16, 16)
                    v = xg_buf[b, i, sl] + e_buf[b, i, sl]
                    msg_buf[b, i, sl] = jnp.maximum(v, 0.0)
                return carry2
            lax.fori_loop(0, _CHUNK, _row, 0)
            pltpu.async_copy(msg_buf.at[b], aggr_sh.at[ibuf.at[j % _IR, 1]],
                             ssem.at[b], add=True)
            @pl.when(j + 2 < _NCHUNK)
            def _():
                _start_loads(j + 2, b)
            @pl.when(j + 6 < _NCHUNK)
            def _():
                _load_idx(j + 6)
        return carry
    lax.fori_loop(0, _NCHUNK // 2, _outer, 0)
    for j in (_NCHUNK - 2, _NCHUNK - 1):
        b = j % 2
        pltpu.make_async_copy(msg_buf.at[b], aggr_sh.at[ibuf.at[j % _IR, 1]],
                              ssem.at[b]).wait()
    plsc.subcore_barrier()

    for z in range(_NFT):
        k = z * _NS + s
        @pl.when(k < _NF)
        def _():
            r0 = k * _FCH
            pltpu.async_copy(aggr_sh.at[pl.ds(r0, _FCH)],
                             part_hbm.at[pl.ds(c * _N + r0, _FCH)],
                             fsem.at[z])
    for z in range(_NFT):
        k = z * _NS + s
        @pl.when(k < _NF)
        def _():
            r0 = k * _FCH
            pltpu.make_async_copy(
                aggr_sh.at[pl.ds(r0, _FCH)],
                part_hbm.at[pl.ds(c * _N + r0, _FCH)], fsem.at[z]).wait()


@functools.cache
def _sc_edge_kernel():
    return pl.kernel(
        _sc_edge_body,
        out_type=jax.ShapeDtypeStruct((2 * _N, _D), jnp.float32),
        mesh=plsc.VectorSubcoreMesh(core_axis_name="c", subcore_axis_name="s",
                                    num_cores=_NC, num_subcores=_NS),
        scratch_types=[
            pltpu.VMEM((_IR, 2, _CHUNK), jnp.int32),
            pltpu.VMEM((2, _CHUNK, _D), jnp.float32),
            pltpu.VMEM((2, _CHUNK, _D), jnp.float32),
            pltpu.VMEM((2, _CHUNK, _D), jnp.float32),
            pltpu.VMEM_SHARED((_N, _D), jnp.float32),
            pltpu.SemaphoreType.DMA((_IR,)),
            pltpu.SemaphoreType.DMA((2,)),
            pltpu.SemaphoreType.DMA((2,)),
            pltpu.SemaphoreType.DMA((2,)),
            pltpu.SemaphoreType.DMA((_NFT,)),
        ],
    )


def _sc_edge(x, e, idx):
    return _sc_edge_kernel()(x, e, idx)


_BE = 8000


def _e_body(ea_ref, w_ref, b_ref, o1_ref, o2_ref, o3_ref):
    t = jnp.dot(ea_ref[...], w_ref[...],
                preferred_element_type=jnp.float32) + b_ref[...]
    for l, o_ref in enumerate((o1_ref, o2_ref, o3_ref)):
        o_ref[...] = t[:, l * _D:(l + 1) * _D]


def _edge_embed(edge_attr, eW, eb):
    return pl.pallas_call(
        _e_body,
        grid=(_E // _BE,),
        in_specs=[
            pl.BlockSpec((_BE, _DE), lambda i: (i, 0)),
            pl.BlockSpec((_DE, 3 * _D), lambda i: (0, 0)),
            pl.BlockSpec((1, 3 * _D), lambda i: (0, 0)),
        ],
        out_specs=[
            pl.BlockSpec((_BE, _D), lambda i: (i, 0)),
            pl.BlockSpec((_BE, _D), lambda i: (i, 0)),
            pl.BlockSpec((_BE, _D), lambda i: (i, 0)),
        ],
        out_shape=[jax.ShapeDtypeStruct((_E, _D), jnp.float32)] * 3,
    )(edge_attr, eW, eb)


def _layer_body(x_ref, part_ref, w1_ref, b1_ref, g_ref, bb_ref,
                w2_ref, b2_ref, o_ref):
    h = x_ref[...] + part_ref[0] + part_ref[1]
    t = jnp.dot(h, w1_ref[...], preferred_element_type=jnp.float32) + b1_ref[...]
    m = jnp.mean(t, axis=0, keepdims=True)
    v = jnp.mean(t * t, axis=0, keepdims=True) - m * m
    t = (t - m) * lax.rsqrt(v + 1e-5) * g_ref[...] + bb_ref[...]
    t = jnp.where(t > 0, t, 0.01 * t)
    t = jnp.dot(t, w2_ref[...], preferred_element_type=jnp.float32) + b2_ref[...]
    t = jnp.where(t > 0, t, 0.01 * t)
    t = jnp.where(t > 0, t, 0.01 * t)
    o_ref[...] = t


def _layer_tc(x, part2, p):
    return pl.pallas_call(
        _layer_body,
        out_shape=jax.ShapeDtypeStruct((_N, _H), jnp.float32),
    )(x, part2, p['W1'], p['b1'][None], p['bn_g'][None], p['bn_b'][None],
      p['W2'], p['b2'][None])


def _head_body(h_ref, batch_ref, wr_ref, br_ref, we_ref, be_ref, o_ref):
    gid = lax.broadcasted_iota(jnp.int32, (_G, 1), 0)
    onehot = (batch_ref[...] == gid).astype(jnp.float32)
    sums = jnp.dot(onehot, h_ref[...], preferred_element_type=jnp.float32)
    cnt = jnp.sum(onehot, axis=1, keepdims=True)
    pooled = sums / jnp.maximum(cnt, 1.0)
    t = jnp.dot(pooled, wr_ref[...], preferred_element_type=jnp.float32) + br_ref[...]
    t = jnp.where(t > 0, t, 0.01 * t)
    o_ref[...] = jnp.dot(t, we_ref[...], preferred_element_type=jnp.float32) + be_ref[...]


def _head_tc(h, batch, p):
    return pl.pallas_call(
        _head_body,
        out_shape=jax.ShapeDtypeStruct((_G, 1), jnp.float32),
    )(h, batch[None], p['Wr1'], p['br1'][None], p['We'], p['be'][None])


def kernel(x, edge_attr, edge_index, batch, params):
    idx = jnp.stack([edge_index[0].reshape(_NC * _NS, _NCHUNK, _CHUNK),
                     edge_index[1].reshape(_NC * _NS, _NCHUNK, _CHUNK)],
                    axis=2)
    layers = params['layers']
    eW = jnp.concatenate([l['eW'] for l in layers], axis=1)
    eb = jnp.concatenate([l['eb'] for l in layers])[None]
    e_all = _edge_embed(edge_attr, eW, eb)
    h = x
    for i in range(3):
        part = _sc_edge(h, e_all[i], idx)
        h = _layer_tc(h, part.reshape(2, _N, _D), layers[i])
    return _head_tc(h, batch, params)
